# baseline (speedup 1.0000x reference)
import jax
import jax.numpy as jnp
from jax import lax
from jax.experimental import pallas as pl
from jax.experimental.pallas import tpu as pltpu
from jax.experimental.pallas import tpu_sc as plsc

BATCH = 4096
HIST = 200
EMBED_DIM = 128
NUM_IDX = BATCH * HIST

NW = 32
PER_W = NUM_IDX // NW
C = 128
NCHUNK = PER_W // C
NBUF = 4

_MESH = plsc.VectorSubcoreMesh(core_axis_name="c", subcore_axis_name="s")


def _ring_kernel(table_hbm, idx_hbm, out_hbm, idx_v, rows_v, isem, gsem, osem):
    wid = lax.axis_index("s") * 2 + lax.axis_index("c")
    base = wid * PER_W

    def idx_copy(g, b):
        return pltpu.make_async_copy(
            idx_hbm.at[pl.ds(base + g * C, C)], idx_v.at[b], isem.at[b])

    def gather_copy(b):
        return pltpu.make_async_copy(
            table_hbm.at[idx_v.at[b]], rows_v.at[b], gsem.at[b])

    def out_copy(g, b):
        return pltpu.make_async_copy(
            rows_v.at[b], out_hbm.at[pl.ds(base + g * C, C)], osem.at[b])

    for b in range(NBUF):
        idx_copy(b, b).start()

    @pl.loop(0, NCHUNK, step=NBUF)
    def _(g0):
        for b in range(NBUF):
            g = g0 + b

            @pl.when(g0 > 0)
            def _():
                out_copy(g - NBUF, b).wait()

            idx_copy(g, b).wait()
            gather_copy(b).start()
            gather_copy(b).wait()
            out_copy(g, b).start()

            @pl.when(g0 + NBUF < NCHUNK)
            def _():
                idx_copy(g + NBUF, b).start()

    for b in range(NBUF):
        out_copy(NCHUNK - NBUF + b, b).wait()


def kernel(x, table):
    idx = x.reshape(NUM_IDX).astype(jnp.int32)
    run = pl.kernel(
        _ring_kernel,
        out_type=jax.ShapeDtypeStruct((NUM_IDX, EMBED_DIM), table.dtype),
        mesh=_MESH,
        scratch_types=[
            pltpu.VMEM((NBUF, C), jnp.int32),
            pltpu.VMEM((NBUF, C, EMBED_DIM), jnp.float32),
            pltpu.SemaphoreType.DMA((NBUF,)),
            pltpu.SemaphoreType.DMA((NBUF,)),
            pltpu.SemaphoreType.DMA((NBUF,)),
        ],
    )
    out = run(table, idx)
    return out.reshape(BATCH, HIST, EMBED_DIM)


# speedup vs baseline: 1.2299x; 1.2299x over previous
import jax
import jax.numpy as jnp
from jax import lax
from jax.experimental import pallas as pl
from jax.experimental.pallas import tpu as pltpu
from jax.experimental.pallas import tpu_sc as plsc

BATCH = 4096
HIST = 200
EMBED_DIM = 128
NUM_IDX = BATCH * HIST

NW = 32
PER_W = NUM_IDX // NW
C = 128
NCHUNK = PER_W // C
NBUF = 4
SKEW = 2

_MESH = plsc.VectorSubcoreMesh(core_axis_name="c", subcore_axis_name="s")


def _ring_kernel(table_hbm, idx_hbm, out_hbm, idx_v, rows_v, isem, gsem, osem):
    wid = lax.axis_index("s") * 2 + lax.axis_index("c")
    base = wid * PER_W

    def idx_copy(g, b):
        return pltpu.make_async_copy(
            idx_hbm.at[pl.ds(base + g * C, C)], idx_v.at[b], isem.at[b])

    def gather_copy(b):
        return pltpu.make_async_copy(
            table_hbm.at[idx_v.at[b]], rows_v.at[b], gsem.at[b])

    def out_copy(g, b):
        return pltpu.make_async_copy(
            rows_v.at[b], out_hbm.at[pl.ds(base + g * C, C)], osem.at[b])

    for b in range(NBUF):
        idx_copy(b, b).start()

    @pl.loop(0, NCHUNK, step=NBUF)
    def _(g0):
        for b in range(NBUF):
            g = g0 + b
            bk = (b - SKEW) % NBUF

            @pl.when(g >= SKEW)
            def _():
                gather_copy(bk).wait()
                out_copy(g - SKEW, bk).start()

                @pl.when(g - SKEW + NBUF < NCHUNK)
                def _():
                    idx_copy(g - SKEW + NBUF, bk).start()

            @pl.when(g >= NBUF)
            def _():
                out_copy(g - NBUF, b).wait()

            idx_copy(g, b).wait()
            gather_copy(b).start()

    for j in range(SKEW):
        g = NCHUNK - SKEW + j
        b = g % NBUF
        gather_copy(b).wait()
        out_copy(g, b).start()

    for b in range(NBUF):
        out_copy(NCHUNK - NBUF + b, b).wait()


def kernel(x, table):
    idx = x.reshape(NUM_IDX).astype(jnp.int32)
    run = pl.kernel(
        _ring_kernel,
        out_type=jax.ShapeDtypeStruct((NUM_IDX, EMBED_DIM), table.dtype),
        mesh=_MESH,
        scratch_types=[
            pltpu.VMEM((NBUF, C), jnp.int32),
            pltpu.VMEM((NBUF, C, EMBED_DIM), jnp.float32),
            pltpu.SemaphoreType.DMA((NBUF,)),
            pltpu.SemaphoreType.DMA((NBUF,)),
            pltpu.SemaphoreType.DMA((NBUF,)),
        ],
    )
    out = run(table, idx)
    return out.reshape(BATCH, HIST, EMBED_DIM)


# speedup vs baseline: 1.2357x; 1.0047x over previous
import jax
import jax.numpy as jnp
from jax import lax
from jax.experimental import pallas as pl
from jax.experimental.pallas import tpu as pltpu
from jax.experimental.pallas import tpu_sc as plsc

BATCH = 4096
HIST = 200
EMBED_DIM = 128
NUM_IDX = BATCH * HIST

NW = 32
PER_W = NUM_IDX // NW
C = 128
NCHUNK = PER_W // C
NBUF = 5
SKEW = 3

_MESH = plsc.VectorSubcoreMesh(core_axis_name="c", subcore_axis_name="s")


def _ring_kernel(table_hbm, idx_hbm, out_hbm, idx_v, rows_v, isem, gsem, osem):
    wid = lax.axis_index("s") * 2 + lax.axis_index("c")
    base = wid * PER_W

    def idx_copy(g, b):
        return pltpu.make_async_copy(
            idx_hbm.at[pl.ds(base + g * C, C)], idx_v.at[b], isem.at[b])

    def gather_copy(b):
        return pltpu.make_async_copy(
            table_hbm.at[idx_v.at[b]], rows_v.at[b], gsem.at[b])

    def out_copy(g, b):
        return pltpu.make_async_copy(
            rows_v.at[b], out_hbm.at[pl.ds(base + g * C, C)], osem.at[b])

    for b in range(NBUF):
        idx_copy(b, b).start()

    @pl.loop(0, NCHUNK, step=NBUF)
    def _(g0):
        for b in range(NBUF):
            g = g0 + b
            bk = (b - SKEW) % NBUF

            @pl.when(g >= SKEW)
            def _():
                gather_copy(bk).wait()
                out_copy(g - SKEW, bk).start()

                @pl.when(g - SKEW + NBUF < NCHUNK)
                def _():
                    idx_copy(g - SKEW + NBUF, bk).start()

            @pl.when(g >= NBUF)
            def _():
                out_copy(g - NBUF, b).wait()

            idx_copy(g, b).wait()
            gather_copy(b).start()

    for j in range(SKEW):
        g = NCHUNK - SKEW + j
        b = g % NBUF
        gather_copy(b).wait()
        out_copy(g, b).start()

    for b in range(NBUF):
        out_copy(NCHUNK - NBUF + b, b).wait()


def kernel(x, table):
    idx = x.reshape(NUM_IDX).astype(jnp.int32)
    run = pl.kernel(
        _ring_kernel,
        out_type=jax.ShapeDtypeStruct((NUM_IDX, EMBED_DIM), table.dtype),
        mesh=_MESH,
        scratch_types=[
            pltpu.VMEM((NBUF, C), jnp.int32),
            pltpu.VMEM((NBUF, C, EMBED_DIM), jnp.float32),
            pltpu.SemaphoreType.DMA((NBUF,)),
            pltpu.SemaphoreType.DMA((NBUF,)),
            pltpu.SemaphoreType.DMA((NBUF,)),
        ],
    )
    out = run(table, idx)
    return out.reshape(BATCH, HIST, EMBED_DIM)
